# baseline (speedup 1.0000x reference)
import functools

import jax
import jax.numpy as jnp
from jax import lax
from jax.experimental import pallas as pl
from jax.experimental.pallas import tpu as pltpu

BATCH = 16
PROMPT_LEN = 128
HIDDEN = 2048
NUM_CODES = 8192
CODE_DIM = 64

TOKENS = BATCH * PROMPT_LEN
TILE = 128
K_CHUNK = 1024


def _body(x_ref, w_enc_ref, b_enc_ref, cb_ref, w_dec_ref, b_dec_ref, out_ref):
    x = x_ref[...]
    w_enc = w_enc_ref[...]
    cb = cb_ref[...]

    z_e = (
        lax.dot_general(x, w_enc, (((1,), (0,)), ((), ())),
                        preferred_element_type=jnp.float32)
        + b_enc_ref[...]
    )

    zz = jnp.sum(z_e * z_e, axis=1, keepdims=True)
    best_d = jnp.full((TILE, 1), jnp.inf, dtype=jnp.float32)
    best_i = jnp.zeros((TILE, 1), dtype=jnp.int32)
    for k in range(NUM_CODES // K_CHUNK):
        cb_c = cb_ref[k * K_CHUNK:(k + 1) * K_CHUNK, :]
        cross = lax.dot_general(z_e, cb_c, (((1,), (1,)), ((), ())),
                                preferred_element_type=jnp.float32)
        cc = jnp.sum(cb_c * cb_c, axis=1)
        d2 = (zz - 2.0 * cross) + cc[None, :]
        local_min = jnp.min(d2, axis=1, keepdims=True)
        liota = lax.broadcasted_iota(jnp.int32, (TILE, K_CHUNK), 1) + k * K_CHUNK
        local_idx = jnp.min(
            jnp.where(d2 == local_min, liota, NUM_CODES),
            axis=1, keepdims=True)
        upd = local_min < best_d
        best_d = jnp.where(upd, local_min, best_d)
        best_i = jnp.where(upd, local_idx, best_i)
    idx = best_i[:, 0]

    k_iota = lax.broadcasted_iota(jnp.int32, (TILE, NUM_CODES), 1)
    onehot = (k_iota == idx[:, None]).astype(jnp.float32)
    z_q = lax.dot_general(onehot, cb, (((1,), (0,)), ((), ())),
                          preferred_element_type=jnp.float32)

    z_q_st = z_e + (z_q - z_e)

    out_ref[...] = (
        lax.dot_general(z_q_st, w_dec_ref[...], (((1,), (0,)), ((), ())),
                        preferred_element_type=jnp.float32)
        + b_dec_ref[...]
    )


@functools.partial(jax.jit, static_argnames=("interpret",))
def kernel(task_des, W_enc, b_enc, codebook, W_dec, b_dec, interpret=False):
    x = task_des.reshape(TOKENS, HIDDEN)
    grid = (TOKENS // TILE,)
    out = pl.pallas_call(
        _body,
        grid=grid,
        in_specs=[
            pl.BlockSpec((TILE, HIDDEN), lambda i: (i, 0)),
            pl.BlockSpec((HIDDEN, CODE_DIM), lambda i: (0, 0)),
            pl.BlockSpec((1, CODE_DIM), lambda i: (0, 0)),
            pl.BlockSpec((NUM_CODES, CODE_DIM), lambda i: (0, 0)),
            pl.BlockSpec((CODE_DIM, HIDDEN), lambda i: (0, 0)),
            pl.BlockSpec((1, HIDDEN), lambda i: (0, 0)),
        ],
        out_specs=pl.BlockSpec((TILE, HIDDEN), lambda i: (i, 0)),
        out_shape=jax.ShapeDtypeStruct((TOKENS, HIDDEN), jnp.float32),
        interpret=interpret,
    )(x, W_enc, b_enc.reshape(1, CODE_DIM), codebook, W_dec,
      b_dec.reshape(1, HIDDEN))
    return out.reshape(BATCH, PROMPT_LEN, HIDDEN)


# speedup vs baseline: 1.0872x; 1.0872x over previous
import functools

import jax
import jax.numpy as jnp
from jax import lax
from jax.experimental import pallas as pl
from jax.experimental.pallas import tpu as pltpu

BATCH = 16
PROMPT_LEN = 128
HIDDEN = 2048
NUM_CODES = 8192
CODE_DIM = 64

TOKENS = BATCH * PROMPT_LEN
TILE = 512
K_CHUNK = 1024


def _body(x_ref, w_enc_ref, b_enc_ref, cb_ref, w_dec_ref, b_dec_ref, out_ref):
    x = x_ref[...]
    w_enc = w_enc_ref[...]
    cb = cb_ref[...]

    z_e = (
        lax.dot_general(x, w_enc, (((1,), (0,)), ((), ())),
                        preferred_element_type=jnp.float32)
        + b_enc_ref[...]
    )

    zz = jnp.sum(z_e * z_e, axis=1, keepdims=True)
    best_d = jnp.full((TILE, 1), jnp.inf, dtype=jnp.float32)
    best_i = jnp.zeros((TILE, 1), dtype=jnp.int32)
    for k in range(NUM_CODES // K_CHUNK):
        cb_c = cb_ref[k * K_CHUNK:(k + 1) * K_CHUNK, :]
        cross = lax.dot_general(z_e, cb_c, (((1,), (1,)), ((), ())),
                                preferred_element_type=jnp.float32)
        cc = jnp.sum(cb_c * cb_c, axis=1)
        d2 = (zz - 2.0 * cross) + cc[None, :]
        local_min = jnp.min(d2, axis=1, keepdims=True)
        liota = lax.broadcasted_iota(jnp.int32, (TILE, K_CHUNK), 1) + k * K_CHUNK
        local_idx = jnp.min(
            jnp.where(d2 == local_min, liota, NUM_CODES),
            axis=1, keepdims=True)
        upd = local_min < best_d
        best_d = jnp.where(upd, local_min, best_d)
        best_i = jnp.where(upd, local_idx, best_i)
    idx = best_i[:, 0]

    k_iota = lax.broadcasted_iota(jnp.int32, (TILE, NUM_CODES), 1)
    onehot = (k_iota == idx[:, None]).astype(jnp.float32)
    z_q = lax.dot_general(onehot, cb, (((1,), (0,)), ((), ())),
                          preferred_element_type=jnp.float32)

    z_q_st = z_e + (z_q - z_e)

    out_ref[...] = (
        lax.dot_general(z_q_st, w_dec_ref[...], (((1,), (0,)), ((), ())),
                        preferred_element_type=jnp.float32)
        + b_dec_ref[...]
    )


@functools.partial(jax.jit, static_argnames=("interpret",))
def kernel(task_des, W_enc, b_enc, codebook, W_dec, b_dec, interpret=False):
    x = task_des.reshape(TOKENS, HIDDEN)
    grid = (TOKENS // TILE,)
    out = pl.pallas_call(
        _body,
        grid=grid,
        in_specs=[
            pl.BlockSpec((TILE, HIDDEN), lambda i: (i, 0)),
            pl.BlockSpec((HIDDEN, CODE_DIM), lambda i: (0, 0)),
            pl.BlockSpec((1, CODE_DIM), lambda i: (0, 0)),
            pl.BlockSpec((NUM_CODES, CODE_DIM), lambda i: (0, 0)),
            pl.BlockSpec((CODE_DIM, HIDDEN), lambda i: (0, 0)),
            pl.BlockSpec((1, HIDDEN), lambda i: (0, 0)),
        ],
        out_specs=pl.BlockSpec((TILE, HIDDEN), lambda i: (i, 0)),
        out_shape=jax.ShapeDtypeStruct((TOKENS, HIDDEN), jnp.float32),
        interpret=interpret,
    )(x, W_enc, b_enc.reshape(1, CODE_DIM), codebook, W_dec,
      b_dec.reshape(1, HIDDEN))
    return out.reshape(BATCH, PROMPT_LEN, HIDDEN)


# speedup vs baseline: 1.2274x; 1.1290x over previous
import functools

import jax
import jax.numpy as jnp
from jax import lax
from jax.experimental import pallas as pl
from jax.experimental.pallas import tpu as pltpu

BATCH = 16
PROMPT_LEN = 128
HIDDEN = 2048
NUM_CODES = 8192
CODE_DIM = 64

TOKENS = BATCH * PROMPT_LEN
TILE = 512
K_CHUNK = 1024


def _body(x_ref, w_enc_ref, b_enc_ref, cb_ref, w_dec_ref, b_dec_ref, out_ref):
    x = x_ref[...]
    w_enc = w_enc_ref[...]
    cb = cb_ref[...]

    z_e = (
        lax.dot_general(x, w_enc, (((1,), (0,)), ((), ())),
                        preferred_element_type=jnp.float32)
        + b_enc_ref[...]
    )

    zz = jnp.sum(z_e * z_e, axis=1, keepdims=True)
    cross = lax.dot_general(z_e, cb, (((1,), (1,)), ((), ())),
                            preferred_element_type=jnp.float32)
    cc = jnp.sum(cb * cb, axis=1)
    d2 = (zz - 2.0 * cross) + cc[None, :]
    idx = jnp.argmin(d2, axis=1).astype(jnp.int32)

    k_iota = lax.broadcasted_iota(jnp.int32, (TILE, NUM_CODES), 1)
    onehot = (k_iota == idx[:, None]).astype(jnp.float32)
    z_q = lax.dot_general(onehot, cb, (((1,), (0,)), ((), ())),
                          preferred_element_type=jnp.float32)

    z_q_st = z_e + (z_q - z_e)

    out_ref[...] = (
        lax.dot_general(z_q_st, w_dec_ref[...], (((1,), (0,)), ((), ())),
                        preferred_element_type=jnp.float32)
        + b_dec_ref[...]
    )


@functools.partial(jax.jit, static_argnames=("interpret",))
def kernel(task_des, W_enc, b_enc, codebook, W_dec, b_dec, interpret=False):
    x = task_des.reshape(TOKENS, HIDDEN)
    grid = (TOKENS // TILE,)
    out = pl.pallas_call(
        _body,
        grid=grid,
        in_specs=[
            pl.BlockSpec((TILE, HIDDEN), lambda i: (i, 0)),
            pl.BlockSpec((HIDDEN, CODE_DIM), lambda i: (0, 0)),
            pl.BlockSpec((1, CODE_DIM), lambda i: (0, 0)),
            pl.BlockSpec((NUM_CODES, CODE_DIM), lambda i: (0, 0)),
            pl.BlockSpec((CODE_DIM, HIDDEN), lambda i: (0, 0)),
            pl.BlockSpec((1, HIDDEN), lambda i: (0, 0)),
        ],
        out_specs=pl.BlockSpec((TILE, HIDDEN), lambda i: (i, 0)),
        out_shape=jax.ShapeDtypeStruct((TOKENS, HIDDEN), jnp.float32),
        interpret=interpret,
    )(x, W_enc, b_enc.reshape(1, CODE_DIM), codebook, W_dec,
      b_dec.reshape(1, HIDDEN))
    return out.reshape(BATCH, PROMPT_LEN, HIDDEN)
